# Initial kernel scaffold; baseline (speedup 1.0000x reference)
import jax
import jax.numpy as jnp
from jax.experimental import pallas as pl


def kernel(parent_struct_feature, parent_geo_feature, params):
    raise NotImplementedError("write your pallas kernel here")


# speedup vs baseline: 39.2273x; 39.2273x over previous
import jax
import jax.numpy as jnp
from jax.experimental import pallas as pl
from jax.experimental.pallas import tpu as pltpu

F_SIZE = 256
H = 256
C = 128
T = 4
ITER = 2
P = 10
S = 57
_NEG = 0.01
_GROUPS = 32
_GSZ = F_SIZE // _GROUPS
_WTILE = 2048
_NWT = (H * C) // _WTILE
_ITILE = 16
_NIT = C // _ITILE
_EPS = 1e-5


def _lrelu(x):
    return jnp.where(x >= 0, x, _NEG * x)


def _dot(a, b):
    return jnp.dot(a, b, preferred_element_type=jnp.float32)


def _parent_kernel(psf_ref, pgf_ref, wp_ref, bp_ref, wg_ref, bg_ref,
                   x_ref, g_ref):
    dn = (((1,), (1,)), ((), ()))
    xw = jax.lax.dot_general(psf_ref[...], wp_ref[...], dn,
                             preferred_element_type=jnp.float32)
    x_ref[0] = _lrelu(xw + bp_ref[0])
    gw = jax.lax.dot_general(pgf_ref[...], wg_ref[...], dn,
                             preferred_element_type=jnp.float32)
    g_ref[0] = _lrelu(gw + bg_ref[0])


def _main_kernel(cf0_ref, g_ref,
                 wgc_ref, bgc_ref, gam_ref, bet_ref,
                 wex_ref, bex_ref, wsem_ref, bsem_ref, wsi_ref, bsi_ref,
                 wel_ref, bel_ref, wee_ref, bee_ref,
                 wi0_ref, wj0_ref, we0_ref, wt0_ref, bn0_ref,
                 wi1_ref, wj1_ref, we1_ref, wt1_ref, bn1_ref,
                 wch_ref, bch_ref, wch2_ref, bch2_ref,
                 cf_out_ref, geo_out_ref, sem_out_ref, si_out_ref,
                 ex_out_ref, ee_out_ref):
    cf0 = cf0_ref[...]

    gc = _dot(g_ref[...], wgc_ref[...]) + bgc_ref[...]
    cidx = jax.lax.broadcasted_iota(jnp.int32, (F_SIZE, _GROUPS), 0)
    gidx = jax.lax.broadcasted_iota(jnp.int32, (F_SIZE, _GROUPS), 1)
    gmat = ((cidx // _GSZ) == gidx).astype(jnp.float32)
    gidx2 = jax.lax.broadcasted_iota(jnp.int32, (_GROUPS, F_SIZE), 0)
    cidx2 = jax.lax.broadcasted_iota(jnp.int32, (_GROUPS, F_SIZE), 1)
    gmat_t = (gidx2 == (cidx2 // _GSZ)).astype(jnp.float32)
    m = _dot(gc, gmat) * (1.0 / _GSZ)
    mb = _dot(m, gmat_t)
    cen = gc - mb
    v = _dot(cen * cen, gmat) * (1.0 / _GSZ)
    vb = _dot(v, gmat_t)
    xn = cen * jax.lax.rsqrt(vb + _EPS)
    geo_out_ref[...] = _lrelu(xn * gam_ref[...] + bet_ref[...])

    ex = _dot(cf0, wex_ref[...]) + bex_ref[...]
    ex_out_ref[...] = ex
    sem_out_ref[...] = _dot(cf0, wsem_ref[...]) + bsem_ref[...]
    si_out_ref[...] = _dot(cf0, wsi_ref[...]) + bsi_ref[...]
    ex_row = jax.lax.dot_general(
        wex_ref[...], cf0, (((0,), (1,)), ((), ())),
        preferred_element_type=jnp.float32) + bex_ref[...]
    expos_col = ex > 0.0
    expos_row = ex_row > 0.0

    bel = bel_ref[...]
    wel = wel_ref[...]

    def el_tile(ti):
        cft = cf0[ti * _ITILE:(ti + 1) * _ITILE, :]
        left = jnp.broadcast_to(cft[:, None, :], (_ITILE, C, H))
        right = jnp.broadcast_to(cf0[None, :, :], (_ITILE, C, H))
        cat = jnp.concatenate([left, right], axis=2).reshape(_ITILE * C,
                                                             2 * H)
        el2 = _lrelu(_dot(cat, wel) + bel)
        return el2.reshape(_ITILE, C, H)

    wee = wee_ref[...]
    bee = bee_ref[...]

    def mp_pass(ti, el3, af, bf, we, wt, bn, store_ee):
        el2 = el3.reshape(_ITILE * C, H)
        e_t = _dot(el2, we).reshape(_ITILE, C, H)
        a_t = af[ti * _ITILE:(ti + 1) * _ITILE, :]
        base = a_t[:, None, :] + bf[None, :, :] + e_t + bn[None, :, :]
        exc_t = expos_col[ti * _ITILE:(ti + 1) * _ITILE, :]
        tile_sum = jnp.zeros((_ITILE, H), dtype=jnp.float32)
        for t in range(T):
            if store_ee:
                cand = jax.lax.dot_general(
                    el3, wee[t, :], (((2,), (0,)), ((), ())),
                    preferred_element_type=jnp.float32) + bee[0, t]
                ee_out_ref[t, ti * _ITILE:(ti + 1) * _ITILE, :] = cand
            else:
                cand = ee_out_ref[t, ti * _ITILE:(ti + 1) * _ITILE, :]
            validf = ((cand > 0.0) & exc_t & expos_row
                      ).astype(jnp.float32)
            msg = _lrelu(base + cand[:, :, None] * wt[t, :][None, None, :])
            msg = msg * validf[:, :, None]
            tile_sum = tile_sum + jnp.sum(msg, axis=1)
        return tile_sum

    def mp_iter(af, bf, we, wt, bn, store_ee):
        return jnp.concatenate(
            [mp_pass(ti, el_tile(ti), af, bf, we, wt, bn, store_ee)
             for ti in range(_NIT)], axis=0)

    af0 = _dot(cf0, wi0_ref[...])
    bf0 = _dot(cf0, wj0_ref[...])
    acc0 = mp_iter(af0, bf0, we0_ref[...], wt0_ref[...], bn0_ref[...], True)

    ee_full = ee_out_ref[...]
    valid_full = ((ee_full > 0.0)
                  & expos_col.reshape(1, C, 1)
                  & expos_row.reshape(1, 1, C))
    has_edges = jnp.any(valid_full)
    cf1 = jnp.where(has_edges, acc0, cf0)

    af1 = _dot(cf1, wi1_ref[...])
    bf1 = _dot(cf1, wj1_ref[...])
    acc1 = mp_iter(af1, bf1, we1_ref[...], wt1_ref[...], bn1_ref[...], False)
    cf2 = jnp.where(has_edges, acc1, cf1)

    cfcat = jnp.concatenate([cf0, cf1, cf2], axis=1)
    hmid = _lrelu(_dot(cfcat, wch_ref[...]) + bch_ref[...])
    cf_out_ref[...] = _lrelu(_dot(hmid, wch2_ref[...]) + bch2_ref[...])


def kernel(parent_struct_feature, parent_geo_feature, params):
    psf = parent_struct_feature
    pgf = parent_geo_feature
    p = params
    f32 = jnp.float32

    wp, bp = p['mlp_parent']
    wg, bg = p['mlp_geo_parent']
    x_flat, g_flat = pl.pallas_call(
        _parent_kernel,
        grid=(_NWT,),
        in_specs=[
            pl.BlockSpec((1, F_SIZE), lambda k: (0, 0)),
            pl.BlockSpec((1, F_SIZE), lambda k: (0, 0)),
            pl.BlockSpec((_WTILE, F_SIZE), lambda k: (k, 0)),
            pl.BlockSpec((1, 1, _WTILE), lambda k: (k, 0, 0)),
            pl.BlockSpec((_WTILE, F_SIZE), lambda k: (k, 0)),
            pl.BlockSpec((1, 1, _WTILE), lambda k: (k, 0, 0)),
        ],
        out_specs=[
            pl.BlockSpec((1, 1, _WTILE), lambda k: (k, 0, 0)),
            pl.BlockSpec((1, 1, _WTILE), lambda k: (k, 0, 0)),
        ],
        out_shape=[
            jax.ShapeDtypeStruct((_NWT, 1, _WTILE), f32),
            jax.ShapeDtypeStruct((_NWT, 1, _WTILE), f32),
        ],
    )(psf, pgf, wp, bp.reshape(_NWT, 1, _WTILE), wg,
      bg.reshape(_NWT, 1, _WTILE))

    cf0 = x_flat.reshape(C, H)
    g2 = g_flat.reshape(C, F_SIZE)

    wgc, bgc = p['mlp_geo_child']
    wex, bex = p['mlp_exists']
    wsem, bsem = p['mlp_sem']
    wsi, bsi = p['mlp_sem_ins']
    wel, bel = p['mlp_edge_latent']
    wee = jnp.concatenate([p['mlp_edge_exists'][t][0] for t in range(T)],
                          axis=0)
    bee = jnp.stack([p['mlp_edge_exists'][t][1][0] for t in range(T)]
                    ).reshape(1, T)
    wn0, bn0 = p['node_edge_op'][0]
    wn1, bn1 = p['node_edge_op'][1]
    wch, bch = p['mlp_child']
    wch2, bch2 = p['mlp_child2']

    outs = pl.pallas_call(
        _main_kernel,
        out_shape=[
            jax.ShapeDtypeStruct((C, F_SIZE), f32),
            jax.ShapeDtypeStruct((C, F_SIZE), f32),
            jax.ShapeDtypeStruct((C, S), f32),
            jax.ShapeDtypeStruct((C, P), f32),
            jax.ShapeDtypeStruct((C, 1), f32),
            jax.ShapeDtypeStruct((T, C, C), f32),
        ],
    )(
        cf0, g2,
        wgc.T, bgc.reshape(1, -1), p['gn_gamma'].reshape(1, -1),
        p['gn_beta'].reshape(1, -1),
        wex.T, bex.reshape(1, 1), wsem.T, bsem.reshape(1, -1),
        wsi.T, bsi.reshape(1, -1),
        wel.T, bel.reshape(1, -1), wee, bee,
        wn0[:, :H].T, wn0[:, H:2 * H].T, wn0[:, 2 * H:3 * H].T,
        wn0[:, 3 * H:].T, bn0.reshape(1, -1),
        wn1[:, :H].T, wn1[:, H:2 * H].T, wn1[:, 2 * H:3 * H].T,
        wn1[:, 3 * H:].T, bn1.reshape(1, -1),
        wch.T, bch.reshape(1, -1), wch2.T, bch2.reshape(1, -1),
    )
    cf, geo, sem, si, exs, ee = outs
    B = psf.shape[0]
    edge_exists = jnp.transpose(ee, (1, 2, 0)).reshape(B, C, C, T)
    return (cf.reshape(B, C, F_SIZE), geo.reshape(B, C, F_SIZE),
            sem.reshape(B, C, S), si.reshape(B, C, P),
            exs.reshape(B, C, 1), edge_exists)


# speedup vs baseline: 39.6922x; 1.0119x over previous
import jax
import jax.numpy as jnp
from jax.experimental import pallas as pl
from jax.experimental.pallas import tpu as pltpu

F_SIZE = 256
H = 256
C = 128
T = 4
ITER = 2
P = 10
S = 57
_NEG = 0.01
_GROUPS = 32
_GSZ = F_SIZE // _GROUPS
_WTILE = 2048
_NWT = (H * C) // _WTILE
_ITILE = 16
_NIT = C // _ITILE
_EPS = 1e-5


def _lrelu(x):
    return jnp.where(x >= 0, x, _NEG * x)


def _dot(a, b):
    return jnp.dot(a, b, preferred_element_type=jnp.float32)


def _parent_kernel(psf_ref, pgf_ref, wp_ref, bp_ref, wg_ref, bg_ref,
                   x_ref, g_ref):
    dn = (((1,), (1,)), ((), ()))
    xw = jax.lax.dot_general(psf_ref[...], wp_ref[...], dn,
                             preferred_element_type=jnp.float32)
    x_ref[0] = _lrelu(xw + bp_ref[0])
    gw = jax.lax.dot_general(pgf_ref[...], wg_ref[...], dn,
                             preferred_element_type=jnp.float32)
    g_ref[0] = _lrelu(gw + bg_ref[0])


def _main_kernel(cf0_ref, g_ref,
                 wgc_ref, bgc_ref, gam_ref, bet_ref,
                 wex_ref, bex_ref, wsem_ref, bsem_ref, wsi_ref, bsi_ref,
                 wel_ref, bel_ref, wee_ref, bee_ref,
                 wi0_ref, wj0_ref, we0_ref, wt0_ref, bn0_ref,
                 wi1_ref, wj1_ref, we1_ref, wt1_ref, bn1_ref,
                 wch_ref, bch_ref, wch2_ref, bch2_ref,
                 cf_out_ref, geo_out_ref, sem_out_ref, si_out_ref,
                 ex_out_ref, ee_out_ref):
    cf0 = cf0_ref[...]

    gc = _dot(g_ref[...], wgc_ref[...]) + bgc_ref[...]
    cidx = jax.lax.broadcasted_iota(jnp.int32, (F_SIZE, _GROUPS), 0)
    gidx = jax.lax.broadcasted_iota(jnp.int32, (F_SIZE, _GROUPS), 1)
    gmat = ((cidx // _GSZ) == gidx).astype(jnp.float32)
    gidx2 = jax.lax.broadcasted_iota(jnp.int32, (_GROUPS, F_SIZE), 0)
    cidx2 = jax.lax.broadcasted_iota(jnp.int32, (_GROUPS, F_SIZE), 1)
    gmat_t = (gidx2 == (cidx2 // _GSZ)).astype(jnp.float32)
    m = _dot(gc, gmat) * (1.0 / _GSZ)
    mb = _dot(m, gmat_t)
    cen = gc - mb
    v = _dot(cen * cen, gmat) * (1.0 / _GSZ)
    vb = _dot(v, gmat_t)
    xn = cen * jax.lax.rsqrt(vb + _EPS)
    geo_out_ref[...] = _lrelu(xn * gam_ref[...] + bet_ref[...])

    ex = _dot(cf0, wex_ref[...]) + bex_ref[...]
    ex_out_ref[...] = ex
    sem_out_ref[...] = _dot(cf0, wsem_ref[...]) + bsem_ref[...]
    si_out_ref[...] = _dot(cf0, wsi_ref[...]) + bsi_ref[...]
    ex_row = jax.lax.dot_general(
        wex_ref[...], cf0, (((0,), (1,)), ((), ())),
        preferred_element_type=jnp.float32) + bex_ref[...]
    expos_col = ex > 0.0
    expos_row = ex_row > 0.0

    bel = bel_ref[...]
    wel = wel_ref[...]
    a1 = _dot(cf0, wel[:H, :])
    a2 = _dot(cf0, wel[H:, :])

    def el_tile(ti):
        a1t = a1[ti * _ITILE:(ti + 1) * _ITILE, :]
        return _lrelu(a1t[:, None, :] + a2[None, :, :] + bel[None, :, :])

    wee = wee_ref[...]
    bee = bee_ref[...]

    def mp_pass(ti, el3, af, bf, we, wt, bn, store_ee):
        el2 = el3.reshape(_ITILE * C, H)
        e_t = _dot(el2, we).reshape(_ITILE, C, H)
        a_t = af[ti * _ITILE:(ti + 1) * _ITILE, :]
        base = a_t[:, None, :] + bf[None, :, :] + e_t + bn[None, :, :]
        exc_t = expos_col[ti * _ITILE:(ti + 1) * _ITILE, :]
        tile_sum = jnp.zeros((_ITILE, H), dtype=jnp.float32)
        for t in range(T):
            if store_ee:
                cand = jax.lax.dot_general(
                    el3, wee[t, :], (((2,), (0,)), ((), ())),
                    preferred_element_type=jnp.float32) + bee[0, t]
                ee_out_ref[t, ti * _ITILE:(ti + 1) * _ITILE, :] = cand
            else:
                cand = ee_out_ref[t, ti * _ITILE:(ti + 1) * _ITILE, :]
            validf = ((cand > 0.0) & exc_t & expos_row
                      ).astype(jnp.float32)
            msg = _lrelu(base + cand[:, :, None] * wt[t, :][None, None, :])
            msg = msg * validf[:, :, None]
            tile_sum = tile_sum + jnp.sum(msg, axis=1)
        return tile_sum

    def mp_iter(af, bf, we, wt, bn, store_ee):
        return jnp.concatenate(
            [mp_pass(ti, el_tile(ti), af, bf, we, wt, bn, store_ee)
             for ti in range(_NIT)], axis=0)

    af0 = _dot(cf0, wi0_ref[...])
    bf0 = _dot(cf0, wj0_ref[...])
    acc0 = mp_iter(af0, bf0, we0_ref[...], wt0_ref[...], bn0_ref[...], True)

    ee_full = ee_out_ref[...]
    valid_full = ((ee_full > 0.0)
                  & expos_col.reshape(1, C, 1)
                  & expos_row.reshape(1, 1, C))
    has_edges = jnp.any(valid_full)
    cf1 = jnp.where(has_edges, acc0, cf0)

    af1 = _dot(cf1, wi1_ref[...])
    bf1 = _dot(cf1, wj1_ref[...])
    acc1 = mp_iter(af1, bf1, we1_ref[...], wt1_ref[...], bn1_ref[...], False)
    cf2 = jnp.where(has_edges, acc1, cf1)

    cfcat = jnp.concatenate([cf0, cf1, cf2], axis=1)
    hmid = _lrelu(_dot(cfcat, wch_ref[...]) + bch_ref[...])
    cf_out_ref[...] = _lrelu(_dot(hmid, wch2_ref[...]) + bch2_ref[...])


def kernel(parent_struct_feature, parent_geo_feature, params):
    psf = parent_struct_feature
    pgf = parent_geo_feature
    p = params
    f32 = jnp.float32

    wp, bp = p['mlp_parent']
    wg, bg = p['mlp_geo_parent']
    x_flat, g_flat = pl.pallas_call(
        _parent_kernel,
        grid=(_NWT,),
        in_specs=[
            pl.BlockSpec((1, F_SIZE), lambda k: (0, 0)),
            pl.BlockSpec((1, F_SIZE), lambda k: (0, 0)),
            pl.BlockSpec((_WTILE, F_SIZE), lambda k: (k, 0)),
            pl.BlockSpec((1, 1, _WTILE), lambda k: (k, 0, 0)),
            pl.BlockSpec((_WTILE, F_SIZE), lambda k: (k, 0)),
            pl.BlockSpec((1, 1, _WTILE), lambda k: (k, 0, 0)),
        ],
        out_specs=[
            pl.BlockSpec((1, 1, _WTILE), lambda k: (k, 0, 0)),
            pl.BlockSpec((1, 1, _WTILE), lambda k: (k, 0, 0)),
        ],
        out_shape=[
            jax.ShapeDtypeStruct((_NWT, 1, _WTILE), f32),
            jax.ShapeDtypeStruct((_NWT, 1, _WTILE), f32),
        ],
    )(psf, pgf, wp, bp.reshape(_NWT, 1, _WTILE), wg,
      bg.reshape(_NWT, 1, _WTILE))

    cf0 = x_flat.reshape(C, H)
    g2 = g_flat.reshape(C, F_SIZE)

    wgc, bgc = p['mlp_geo_child']
    wex, bex = p['mlp_exists']
    wsem, bsem = p['mlp_sem']
    wsi, bsi = p['mlp_sem_ins']
    wel, bel = p['mlp_edge_latent']
    wee = jnp.concatenate([p['mlp_edge_exists'][t][0] for t in range(T)],
                          axis=0)
    bee = jnp.stack([p['mlp_edge_exists'][t][1][0] for t in range(T)]
                    ).reshape(1, T)
    wn0, bn0 = p['node_edge_op'][0]
    wn1, bn1 = p['node_edge_op'][1]
    wch, bch = p['mlp_child']
    wch2, bch2 = p['mlp_child2']

    outs = pl.pallas_call(
        _main_kernel,
        out_shape=[
            jax.ShapeDtypeStruct((C, F_SIZE), f32),
            jax.ShapeDtypeStruct((C, F_SIZE), f32),
            jax.ShapeDtypeStruct((C, S), f32),
            jax.ShapeDtypeStruct((C, P), f32),
            jax.ShapeDtypeStruct((C, 1), f32),
            jax.ShapeDtypeStruct((T, C, C), f32),
        ],
    )(
        cf0, g2,
        wgc.T, bgc.reshape(1, -1), p['gn_gamma'].reshape(1, -1),
        p['gn_beta'].reshape(1, -1),
        wex.T, bex.reshape(1, 1), wsem.T, bsem.reshape(1, -1),
        wsi.T, bsi.reshape(1, -1),
        wel.T, bel.reshape(1, -1), wee, bee,
        wn0[:, :H].T, wn0[:, H:2 * H].T, wn0[:, 2 * H:3 * H].T,
        wn0[:, 3 * H:].T, bn0.reshape(1, -1),
        wn1[:, :H].T, wn1[:, H:2 * H].T, wn1[:, 2 * H:3 * H].T,
        wn1[:, 3 * H:].T, bn1.reshape(1, -1),
        wch.T, bch.reshape(1, -1), wch2.T, bch2.reshape(1, -1),
    )
    cf, geo, sem, si, exs, ee = outs
    B = psf.shape[0]
    edge_exists = jnp.transpose(ee, (1, 2, 0)).reshape(B, C, C, T)
    return (cf.reshape(B, C, F_SIZE), geo.reshape(B, C, F_SIZE),
            sem.reshape(B, C, S), si.reshape(B, C, P),
            exs.reshape(B, C, 1), edge_exists)


# speedup vs baseline: 40.4330x; 1.0187x over previous
import jax
import jax.numpy as jnp
from jax.experimental import pallas as pl
from jax.experimental.pallas import tpu as pltpu

F_SIZE = 256
H = 256
C = 128
T = 4
ITER = 2
P = 10
S = 57
_NEG = 0.01
_GROUPS = 32
_GSZ = F_SIZE // _GROUPS
_WTILE = 2048
_NWT = (H * C) // _WTILE
_ITILE = 16
_NIT = C // _ITILE
_EPS = 1e-5


def _lrelu(x):
    return jnp.where(x >= 0, x, _NEG * x)


def _dot(a, b):
    return jnp.dot(a, b, preferred_element_type=jnp.float32)


def _bdot(a, b):
    return jnp.dot(a.astype(jnp.bfloat16), b,
                   preferred_element_type=jnp.float32)


def _parent_kernel(psf_ref, pgf_ref, wp_ref, bp_ref, wg_ref, bg_ref,
                   x_ref, g_ref):
    dn = (((1,), (1,)), ((), ()))
    xw = jax.lax.dot_general(psf_ref[...], wp_ref[...], dn,
                             preferred_element_type=jnp.float32)
    x_ref[0] = _lrelu(xw + bp_ref[0])
    gw = jax.lax.dot_general(pgf_ref[...], wg_ref[...], dn,
                             preferred_element_type=jnp.float32)
    g_ref[0] = _lrelu(gw + bg_ref[0])


def _main_kernel(cf0_ref, g_ref,
                 wgc_ref, bgc_ref, gam_ref, bet_ref,
                 wex_ref, bex_ref, wsem_ref, bsem_ref, wsi_ref, bsi_ref,
                 wel_ref, bel_ref, wee_ref, bee_ref,
                 wi0_ref, wj0_ref, we0_ref, wt0_ref, bn0_ref,
                 wi1_ref, wj1_ref, we1_ref, wt1_ref, bn1_ref,
                 wch_ref, bch_ref, wch2_ref, bch2_ref,
                 cf_out_ref, geo_out_ref, sem_out_ref, si_out_ref,
                 ex_out_ref, ee_out_ref):
    cf0 = cf0_ref[...]

    gc = _dot(g_ref[...], wgc_ref[...]) + bgc_ref[...]
    cidx = jax.lax.broadcasted_iota(jnp.int32, (F_SIZE, _GROUPS), 0)
    gidx = jax.lax.broadcasted_iota(jnp.int32, (F_SIZE, _GROUPS), 1)
    gmat = ((cidx // _GSZ) == gidx).astype(jnp.float32)
    gidx2 = jax.lax.broadcasted_iota(jnp.int32, (_GROUPS, F_SIZE), 0)
    cidx2 = jax.lax.broadcasted_iota(jnp.int32, (_GROUPS, F_SIZE), 1)
    gmat_t = (gidx2 == (cidx2 // _GSZ)).astype(jnp.float32)
    m = _dot(gc, gmat) * (1.0 / _GSZ)
    mb = _dot(m, gmat_t)
    cen = gc - mb
    v = _dot(cen * cen, gmat) * (1.0 / _GSZ)
    vb = _dot(v, gmat_t)
    xn = cen * jax.lax.rsqrt(vb + _EPS)
    geo_out_ref[...] = _lrelu(xn * gam_ref[...] + bet_ref[...])

    ex = _dot(cf0, wex_ref[...]) + bex_ref[...]
    ex_out_ref[...] = ex
    sem_out_ref[...] = _dot(cf0, wsem_ref[...]) + bsem_ref[...]
    si_out_ref[...] = _dot(cf0, wsi_ref[...]) + bsi_ref[...]
    ex_row = jax.lax.dot_general(
        wex_ref[...], cf0, (((0,), (1,)), ((), ())),
        preferred_element_type=jnp.float32) + bex_ref[...]
    expos_col = ex > 0.0
    expos_row = ex_row > 0.0

    bel = bel_ref[...]
    wel = wel_ref[...]
    a1 = _bdot(cf0, wel[:H, :])
    a2 = _bdot(cf0, wel[H:, :])

    def el_tile(ti):
        a1t = a1[ti * _ITILE:(ti + 1) * _ITILE, :]
        return _lrelu(a1t[:, None, :] + a2[None, :, :] + bel[None, :, :])

    wee = wee_ref[...]
    bee = bee_ref[...]

    def mp_pass(ti, el3, af, bf, we, wt, bn, store_ee):
        el3b = el3.astype(jnp.bfloat16)
        e_t = _dot(el3b.reshape(_ITILE * C, H), we).reshape(_ITILE, C, H)
        a_t = af[ti * _ITILE:(ti + 1) * _ITILE, :]
        base = a_t[:, None, :] + bf[None, :, :] + e_t + bn[None, :, :]
        exc_t = expos_col[ti * _ITILE:(ti + 1) * _ITILE, :]
        el3r = el3b.astype(jnp.float32)
        tile_sum = jnp.zeros((_ITILE, H), dtype=jnp.float32)
        for t in range(T):
            if store_ee:
                cand = jax.lax.dot_general(
                    el3r, wee[t, :], (((2,), (0,)), ((), ())),
                    preferred_element_type=jnp.float32) + bee[0, t]
                ee_out_ref[t, ti * _ITILE:(ti + 1) * _ITILE, :] = cand
            else:
                cand = ee_out_ref[t, ti * _ITILE:(ti + 1) * _ITILE, :]
            validf = ((cand > 0.0) & exc_t & expos_row
                      ).astype(jnp.float32)
            candb = cand.astype(jnp.bfloat16).astype(jnp.float32)
            msg = _lrelu(base + candb[:, :, None] * wt[t, :][None, None, :])
            msg = msg * validf[:, :, None]
            tile_sum = tile_sum + jnp.sum(msg, axis=1)
        return tile_sum

    def mp_iter(af, bf, we, wt, bn, store_ee):
        return jnp.concatenate(
            [mp_pass(ti, el_tile(ti), af, bf, we, wt, bn, store_ee)
             for ti in range(_NIT)], axis=0)

    af0 = _bdot(cf0, wi0_ref[...])
    bf0 = _bdot(cf0, wj0_ref[...])
    acc0 = mp_iter(af0, bf0, we0_ref[...], wt0_ref[...], bn0_ref[...], True)

    ee_full = ee_out_ref[...]
    valid_full = ((ee_full > 0.0)
                  & expos_col.reshape(1, C, 1)
                  & expos_row.reshape(1, 1, C))
    has_edges = jnp.any(valid_full)
    cf1 = jnp.where(has_edges, acc0, cf0)

    af1 = _bdot(cf1, wi1_ref[...])
    bf1 = _bdot(cf1, wj1_ref[...])
    acc1 = mp_iter(af1, bf1, we1_ref[...], wt1_ref[...], bn1_ref[...], False)
    cf2 = jnp.where(has_edges, acc1, cf1)

    cfcat = jnp.concatenate([cf0, cf1, cf2], axis=1)
    hmid = _lrelu(_bdot(cfcat, wch_ref[...]) + bch_ref[...])
    cf_out_ref[...] = _lrelu(_bdot(hmid, wch2_ref[...]) + bch2_ref[...])


def kernel(parent_struct_feature, parent_geo_feature, params):
    psf = parent_struct_feature
    pgf = parent_geo_feature
    p = params
    f32 = jnp.float32
    bf16 = jnp.bfloat16

    wp, bp = p['mlp_parent']
    wg, bg = p['mlp_geo_parent']
    x_flat, g_flat = pl.pallas_call(
        _parent_kernel,
        grid=(_NWT,),
        in_specs=[
            pl.BlockSpec((1, F_SIZE), lambda k: (0, 0)),
            pl.BlockSpec((1, F_SIZE), lambda k: (0, 0)),
            pl.BlockSpec((_WTILE, F_SIZE), lambda k: (k, 0)),
            pl.BlockSpec((1, 1, _WTILE), lambda k: (k, 0, 0)),
            pl.BlockSpec((_WTILE, F_SIZE), lambda k: (k, 0)),
            pl.BlockSpec((1, 1, _WTILE), lambda k: (k, 0, 0)),
        ],
        out_specs=[
            pl.BlockSpec((1, 1, _WTILE), lambda k: (k, 0, 0)),
            pl.BlockSpec((1, 1, _WTILE), lambda k: (k, 0, 0)),
        ],
        out_shape=[
            jax.ShapeDtypeStruct((_NWT, 1, _WTILE), f32),
            jax.ShapeDtypeStruct((_NWT, 1, _WTILE), f32),
        ],
    )(psf, pgf, wp, bp.reshape(_NWT, 1, _WTILE), wg,
      bg.reshape(_NWT, 1, _WTILE))

    cf0 = x_flat.reshape(C, H)
    g2 = g_flat.reshape(C, F_SIZE)

    wgc, bgc = p['mlp_geo_child']
    wex, bex = p['mlp_exists']
    wsem, bsem = p['mlp_sem']
    wsi, bsi = p['mlp_sem_ins']
    wel, bel = p['mlp_edge_latent']
    wee = jnp.concatenate([p['mlp_edge_exists'][t][0] for t in range(T)],
                          axis=0)
    bee = jnp.stack([p['mlp_edge_exists'][t][1][0] for t in range(T)]
                    ).reshape(1, T)
    wn0, bn0 = p['node_edge_op'][0]
    wn1, bn1 = p['node_edge_op'][1]
    wch, bch = p['mlp_child']
    wch2, bch2 = p['mlp_child2']

    outs = pl.pallas_call(
        _main_kernel,
        out_shape=[
            jax.ShapeDtypeStruct((C, F_SIZE), f32),
            jax.ShapeDtypeStruct((C, F_SIZE), f32),
            jax.ShapeDtypeStruct((C, S), f32),
            jax.ShapeDtypeStruct((C, P), f32),
            jax.ShapeDtypeStruct((C, 1), f32),
            jax.ShapeDtypeStruct((T, C, C), f32),
        ],
    )(
        cf0, g2,
        wgc.T, bgc.reshape(1, -1), p['gn_gamma'].reshape(1, -1),
        p['gn_beta'].reshape(1, -1),
        wex.T, bex.reshape(1, 1), wsem.T, bsem.reshape(1, -1),
        wsi.T, bsi.reshape(1, -1),
        wel.T.astype(bf16), bel.reshape(1, -1),
        wee.astype(bf16).astype(f32), bee,
        wn0[:, :H].T.astype(bf16), wn0[:, H:2 * H].T.astype(bf16),
        wn0[:, 2 * H:3 * H].T.astype(bf16),
        wn0[:, 3 * H:].T.astype(bf16), bn0.reshape(1, -1),
        wn1[:, :H].T.astype(bf16), wn1[:, H:2 * H].T.astype(bf16),
        wn1[:, 2 * H:3 * H].T.astype(bf16),
        wn1[:, 3 * H:].T.astype(bf16), bn1.reshape(1, -1),
        wch.T.astype(bf16), bch.reshape(1, -1),
        wch2.T.astype(bf16), bch2.reshape(1, -1),
    )
    cf, geo, sem, si, exs, ee = outs
    B = psf.shape[0]
    edge_exists = jnp.transpose(ee, (1, 2, 0)).reshape(B, C, C, T)
    return (cf.reshape(B, C, F_SIZE), geo.reshape(B, C, F_SIZE),
            sem.reshape(B, C, S), si.reshape(B, C, P),
            exs.reshape(B, C, 1), edge_exists)


# speedup vs baseline: 42.4838x; 1.0507x over previous
import jax
import jax.numpy as jnp
from jax.experimental import pallas as pl
from jax.experimental.pallas import tpu as pltpu

F_SIZE = 256
H = 256
C = 128
T = 4
ITER = 2
P = 10
S = 57
_NEG = 0.01
_GROUPS = 32
_GSZ = F_SIZE // _GROUPS
_WTILE = 4096
_NWT = (H * C) // _WTILE
_ITILE = 16
_NIT = C // _ITILE
_EPS = 1e-5


def _lrelu(x):
    return jnp.maximum(x, _NEG * x)


def _dot(a, b):
    return jnp.dot(a, b, preferred_element_type=jnp.float32)


def _bdot(a, b):
    return jnp.dot(a.astype(jnp.bfloat16), b,
                   preferred_element_type=jnp.float32)


def _parent_kernel(psf_ref, pgf_ref, wp_ref, bp_ref, wg_ref, bg_ref,
                   x_ref, g_ref):
    dn = (((1,), (1,)), ((), ()))
    xw = jax.lax.dot_general(psf_ref[...], wp_ref[...], dn,
                             preferred_element_type=jnp.float32)
    x_ref[0] = _lrelu(xw + bp_ref[0])
    gw = jax.lax.dot_general(pgf_ref[...], wg_ref[...], dn,
                             preferred_element_type=jnp.float32)
    g_ref[0] = _lrelu(gw + bg_ref[0])


def _main_kernel(cf0_ref, g_ref,
                 wgc_ref, bgc_ref, gam_ref, bet_ref,
                 wex_ref, bex_ref, wsem_ref, bsem_ref, wsi_ref, bsi_ref,
                 wel_ref, bel_ref, wee_ref, bee_ref,
                 wi0_ref, wj0_ref, we0_ref, wt0_ref, bn0_ref,
                 wi1_ref, wj1_ref, we1_ref, wt1_ref, bn1_ref,
                 wch_ref, bch_ref, wch2_ref, bch2_ref,
                 cf_out_ref, geo_out_ref, sem_out_ref, si_out_ref,
                 ex_out_ref, ee_out_ref):
    cf0 = cf0_ref[...]

    gc = _dot(g_ref[...], wgc_ref[...]) + bgc_ref[...]
    cidx = jax.lax.broadcasted_iota(jnp.int32, (F_SIZE, _GROUPS), 0)
    gidx = jax.lax.broadcasted_iota(jnp.int32, (F_SIZE, _GROUPS), 1)
    gmat = ((cidx // _GSZ) == gidx).astype(jnp.float32)
    gidx2 = jax.lax.broadcasted_iota(jnp.int32, (_GROUPS, F_SIZE), 0)
    cidx2 = jax.lax.broadcasted_iota(jnp.int32, (_GROUPS, F_SIZE), 1)
    gmat_t = (gidx2 == (cidx2 // _GSZ)).astype(jnp.float32)
    m = _dot(gc, gmat) * (1.0 / _GSZ)
    mb = _dot(m, gmat_t)
    cen = gc - mb
    v = _dot(cen * cen, gmat) * (1.0 / _GSZ)
    vb = _dot(v, gmat_t)
    xn = cen * jax.lax.rsqrt(vb + _EPS)
    geo_out_ref[...] = _lrelu(xn * gam_ref[...] + bet_ref[...])

    ex = _dot(cf0, wex_ref[...]) + bex_ref[...]
    ex_out_ref[...] = ex
    sem_out_ref[...] = _dot(cf0, wsem_ref[...]) + bsem_ref[...]
    si_out_ref[...] = _dot(cf0, wsi_ref[...]) + bsi_ref[...]
    ex_row = jax.lax.dot_general(
        wex_ref[...], cf0, (((0,), (1,)), ((), ())),
        preferred_element_type=jnp.float32) + bex_ref[...]
    expos_col = ex > 0.0
    expos_row = ex_row > 0.0

    bel = bel_ref[...]
    wel = wel_ref[...]
    a1 = _bdot(cf0, wel[:H, :])
    a2 = _bdot(cf0, wel[H:, :])

    def el_tile(ti):
        a1t = a1[ti * _ITILE:(ti + 1) * _ITILE, :]
        return _lrelu(a1t[:, None, :] + a2[None, :, :] + bel[None, :, :])

    wee = wee_ref[...]
    bee = bee_ref[...]

    def mp_pass(ti, el3, af, bf, we, wt, bn, store_ee):
        el3b = el3.astype(jnp.bfloat16)
        e_t = _dot(el3b.reshape(_ITILE * C, H), we).reshape(_ITILE, C, H)
        a_t = af[ti * _ITILE:(ti + 1) * _ITILE, :]
        base = a_t[:, None, :] + bf[None, :, :] + e_t + bn[None, :, :]
        exc_t = expos_col[ti * _ITILE:(ti + 1) * _ITILE, :]
        el3r = el3b.astype(jnp.float32)
        tile_sum = jnp.zeros((_ITILE, H), dtype=jnp.float32)
        for t in range(T):
            if store_ee:
                cand = jax.lax.dot_general(
                    el3r, wee[t, :], (((2,), (0,)), ((), ())),
                    preferred_element_type=jnp.float32) + bee[0, t]
                ee_out_ref[t, ti * _ITILE:(ti + 1) * _ITILE, :] = cand
            else:
                cand = ee_out_ref[t, ti * _ITILE:(ti + 1) * _ITILE, :]
            validf = ((cand > 0.0) & exc_t & expos_row
                      ).astype(jnp.float32)
            candb = cand.astype(jnp.bfloat16).astype(jnp.float32)
            msg = _lrelu(base + candb[:, :, None] * wt[t, :][None, None, :])
            msg = msg * validf[:, :, None]
            tile_sum = tile_sum + jnp.sum(msg, axis=1)
        return tile_sum

    def mp_iter(af, bf, we, wt, bn, store_ee):
        return jnp.concatenate(
            [mp_pass(ti, el_tile(ti), af, bf, we, wt, bn, store_ee)
             for ti in range(_NIT)], axis=0)

    af0 = _bdot(cf0, wi0_ref[...])
    bf0 = _bdot(cf0, wj0_ref[...])
    acc0 = mp_iter(af0, bf0, we0_ref[...], wt0_ref[...], bn0_ref[...], True)

    ee_full = ee_out_ref[...]
    valid_full = ((ee_full > 0.0)
                  & expos_col.reshape(1, C, 1)
                  & expos_row.reshape(1, 1, C))
    has_edges = jnp.any(valid_full)
    cf1 = jnp.where(has_edges, acc0, cf0)

    af1 = _bdot(cf1, wi1_ref[...])
    bf1 = _bdot(cf1, wj1_ref[...])
    acc1 = mp_iter(af1, bf1, we1_ref[...], wt1_ref[...], bn1_ref[...], False)
    cf2 = jnp.where(has_edges, acc1, cf1)

    cfcat = jnp.concatenate([cf0, cf1, cf2], axis=1)
    hmid = _lrelu(_bdot(cfcat, wch_ref[...]) + bch_ref[...])
    cf_out_ref[...] = _lrelu(_bdot(hmid, wch2_ref[...]) + bch2_ref[...])


def kernel(parent_struct_feature, parent_geo_feature, params):
    psf = parent_struct_feature
    pgf = parent_geo_feature
    p = params
    f32 = jnp.float32
    bf16 = jnp.bfloat16

    wp, bp = p['mlp_parent']
    wg, bg = p['mlp_geo_parent']
    x_flat, g_flat = pl.pallas_call(
        _parent_kernel,
        grid=(_NWT,),
        in_specs=[
            pl.BlockSpec((1, F_SIZE), lambda k: (0, 0)),
            pl.BlockSpec((1, F_SIZE), lambda k: (0, 0)),
            pl.BlockSpec((_WTILE, F_SIZE), lambda k: (k, 0)),
            pl.BlockSpec((1, 1, _WTILE), lambda k: (k, 0, 0)),
            pl.BlockSpec((_WTILE, F_SIZE), lambda k: (k, 0)),
            pl.BlockSpec((1, 1, _WTILE), lambda k: (k, 0, 0)),
        ],
        out_specs=[
            pl.BlockSpec((1, 1, _WTILE), lambda k: (k, 0, 0)),
            pl.BlockSpec((1, 1, _WTILE), lambda k: (k, 0, 0)),
        ],
        out_shape=[
            jax.ShapeDtypeStruct((_NWT, 1, _WTILE), f32),
            jax.ShapeDtypeStruct((_NWT, 1, _WTILE), f32),
        ],
    )(psf, pgf, wp, bp.reshape(_NWT, 1, _WTILE), wg,
      bg.reshape(_NWT, 1, _WTILE))

    cf0 = x_flat.reshape(C, H)
    g2 = g_flat.reshape(C, F_SIZE)

    wgc, bgc = p['mlp_geo_child']
    wex, bex = p['mlp_exists']
    wsem, bsem = p['mlp_sem']
    wsi, bsi = p['mlp_sem_ins']
    wel, bel = p['mlp_edge_latent']
    wee = jnp.concatenate([p['mlp_edge_exists'][t][0] for t in range(T)],
                          axis=0)
    bee = jnp.stack([p['mlp_edge_exists'][t][1][0] for t in range(T)]
                    ).reshape(1, T)
    wn0, bn0 = p['node_edge_op'][0]
    wn1, bn1 = p['node_edge_op'][1]
    wch, bch = p['mlp_child']
    wch2, bch2 = p['mlp_child2']

    outs = pl.pallas_call(
        _main_kernel,
        out_shape=[
            jax.ShapeDtypeStruct((C, F_SIZE), f32),
            jax.ShapeDtypeStruct((C, F_SIZE), f32),
            jax.ShapeDtypeStruct((C, S), f32),
            jax.ShapeDtypeStruct((C, P), f32),
            jax.ShapeDtypeStruct((C, 1), f32),
            jax.ShapeDtypeStruct((T, C, C), f32),
        ],
    )(
        cf0, g2,
        wgc.T, bgc.reshape(1, -1), p['gn_gamma'].reshape(1, -1),
        p['gn_beta'].reshape(1, -1),
        wex.T, bex.reshape(1, 1), wsem.T, bsem.reshape(1, -1),
        wsi.T, bsi.reshape(1, -1),
        wel.T.astype(bf16), bel.reshape(1, -1),
        wee.astype(bf16).astype(f32), bee,
        wn0[:, :H].T.astype(bf16), wn0[:, H:2 * H].T.astype(bf16),
        wn0[:, 2 * H:3 * H].T.astype(bf16),
        wn0[:, 3 * H:].T.astype(bf16), bn0.reshape(1, -1),
        wn1[:, :H].T.astype(bf16), wn1[:, H:2 * H].T.astype(bf16),
        wn1[:, 2 * H:3 * H].T.astype(bf16),
        wn1[:, 3 * H:].T.astype(bf16), bn1.reshape(1, -1),
        wch.T.astype(bf16), bch.reshape(1, -1),
        wch2.T.astype(bf16), bch2.reshape(1, -1),
    )
    cf, geo, sem, si, exs, ee = outs
    B = psf.shape[0]
    edge_exists = jnp.transpose(ee, (1, 2, 0)).reshape(B, C, C, T)
    return (cf.reshape(B, C, F_SIZE), geo.reshape(B, C, F_SIZE),
            sem.reshape(B, C, S), si.reshape(B, C, P),
            exs.reshape(B, C, 1), edge_exists)
